# baseline (speedup 1.0000x reference)
import jax
import jax.numpy as jnp
from jax.experimental import pallas as pl
from jax.experimental.pallas import tpu as pltpu

D_MODEL_K = 2048
N_EXPERTS_K = 8
ROUTER_HIDDEN_K = 64
B_K, S_K = 4, 4096
CHUNK = 512
NT = S_K // CHUNK


def _body(x_ref, W1_ref, b1_ref, W2_ref, b2_ref, eb_ref, scale_ref,
          out_ref, xbuf, acc, biasbuf):
    p = pl.program_id(1)
    t = pl.program_id(2)

    @pl.when(p == 0)
    def _read_phase():
        chunk = x_ref[0]
        xbuf[pl.ds(t * CHUNK, CHUNK), :] = chunk
        ones = jnp.ones((1, CHUNK), dtype=jnp.float32)
        partial = jnp.dot(ones, chunk,
                          preferred_element_type=jnp.float32)

        @pl.when(t == 0)
        def _():
            acc[...] = partial

        @pl.when(t > 0)
        def _():
            acc[...] = acc[...] + partial

        @pl.when(t == NT - 1)
        def _router():
            mean = acc[...] * (1.0 / S_K)
            h = jnp.dot(mean, W1_ref[...],
                        preferred_element_type=jnp.float32) + b1_ref[...]
            h = jnp.maximum(h, 0.0)
            logits = jnp.dot(h, W2_ref[...],
                             preferred_element_type=jnp.float32) + b2_ref[...]
            idx = jax.lax.broadcasted_iota(jnp.int32, (1, N_EXPERTS_K), 1)
            m1 = jnp.max(logits, axis=1, keepdims=True)
            i1 = jnp.min(jnp.where(logits == m1, idx, N_EXPERTS_K),
                         axis=1, keepdims=True)
            l2 = jnp.where(idx == i1, -1e30, logits)
            m2 = jnp.max(l2, axis=1, keepdims=True)
            i2 = jnp.min(jnp.where(l2 == m2, idx, N_EXPERTS_K),
                         axis=1, keepdims=True)
            e2 = jnp.exp(m2 - m1)
            denom = 1.0 + e2
            wvec = jnp.where(idx == i1, 1.0 / denom,
                             jnp.where(idx == i2, e2 / denom, 0.0))
            comb = jnp.dot(wvec, eb_ref[...],
                           preferred_element_type=jnp.float32)
            biasbuf[...] = comb * scale_ref[...]

    @pl.when(p == 1)
    def _write_phase():
        out_ref[0] = xbuf[pl.ds(t * CHUNK, CHUNK), :] + biasbuf[...]


@jax.jit
def _run(x, W1, b1, W2, b2, expert_biases, bias_scale):
    grid = (B_K, 2, NT)
    return pl.pallas_call(
        _body,
        grid=grid,
        in_specs=[
            pl.BlockSpec(
                (1, CHUNK, D_MODEL_K),
                lambda b, p, t: (b, jnp.where(p == 0, t, NT - 1), 0)),
            pl.BlockSpec((D_MODEL_K, ROUTER_HIDDEN_K), lambda b, p, t: (0, 0)),
            pl.BlockSpec((1, ROUTER_HIDDEN_K), lambda b, p, t: (0, 0)),
            pl.BlockSpec((ROUTER_HIDDEN_K, N_EXPERTS_K), lambda b, p, t: (0, 0)),
            pl.BlockSpec((1, N_EXPERTS_K), lambda b, p, t: (0, 0)),
            pl.BlockSpec((N_EXPERTS_K, D_MODEL_K), lambda b, p, t: (0, 0)),
            pl.BlockSpec((1, 1), lambda b, p, t: (0, 0)),
        ],
        out_specs=pl.BlockSpec(
            (1, CHUNK, D_MODEL_K),
            lambda b, p, t: (b, jnp.where(p == 1, t, 0), 0)),
        out_shape=jax.ShapeDtypeStruct((B_K, S_K, D_MODEL_K), jnp.float32),
        scratch_shapes=[
            pltpu.VMEM((S_K, D_MODEL_K), jnp.float32),
            pltpu.VMEM((1, D_MODEL_K), jnp.float32),
            pltpu.VMEM((1, D_MODEL_K), jnp.float32),
        ],
    )(x, W1, b1.reshape(1, -1), W2, b2.reshape(1, -1),
      expert_biases, bias_scale.reshape(1, 1))


def kernel(x, W1, b1, W2, b2, expert_biases, bias_scale):
    return _run(x, W1, b1, W2, b2, expert_biases, bias_scale)


# speedup vs baseline: 1.0510x; 1.0510x over previous
import jax
import jax.numpy as jnp
from jax.experimental import pallas as pl
from jax.experimental.pallas import tpu as pltpu

D_MODEL_K = 2048
N_EXPERTS_K = 8
ROUTER_HIDDEN_K = 64
B_K, S_K = 4, 4096
CHUNK = 512
NT = S_K // CHUNK


def _body(x_ref, W1_ref, b1_ref, W2_ref, b2_ref, eb_ref, scale_ref,
          out_ref, xbuf, acc, biasbuf):
    p = pl.program_id(1)
    t = pl.program_id(2)

    @pl.when(p == 0)
    def _read_phase():
        chunk = x_ref[0]
        xbuf[pl.ds(t * CHUNK, CHUNK), :] = chunk
        partial = jnp.sum(chunk, axis=0, keepdims=True)

        @pl.when(t == 0)
        def _():
            acc[...] = partial

        @pl.when(t > 0)
        def _():
            acc[...] = acc[...] + partial

        @pl.when(t == NT - 1)
        def _router():
            mean = acc[...] * (1.0 / S_K)
            h = jnp.dot(mean, W1_ref[...],
                        preferred_element_type=jnp.float32) + b1_ref[...]
            h = jnp.maximum(h, 0.0)
            logits = jnp.dot(h, W2_ref[...],
                             preferred_element_type=jnp.float32) + b2_ref[...]
            idx = jax.lax.broadcasted_iota(jnp.int32, (1, N_EXPERTS_K), 1)
            m1 = jnp.max(logits, axis=1, keepdims=True)
            i1 = jnp.min(jnp.where(logits == m1, idx, N_EXPERTS_K),
                         axis=1, keepdims=True)
            l2 = jnp.where(idx == i1, -1e30, logits)
            m2 = jnp.max(l2, axis=1, keepdims=True)
            i2 = jnp.min(jnp.where(l2 == m2, idx, N_EXPERTS_K),
                         axis=1, keepdims=True)
            e2 = jnp.exp(m2 - m1)
            denom = 1.0 + e2
            wvec = jnp.where(idx == i1, 1.0 / denom,
                             jnp.where(idx == i2, e2 / denom, 0.0))
            comb = jnp.dot(wvec, eb_ref[...],
                           preferred_element_type=jnp.float32)
            biasbuf[...] = comb * scale_ref[...]

    @pl.when(p == 1)
    def _write_phase():
        out_ref[0] = xbuf[pl.ds(t * CHUNK, CHUNK), :] + biasbuf[...]


@jax.jit
def _run(x, W1, b1, W2, b2, expert_biases, bias_scale):
    grid = (B_K, 2, NT)
    return pl.pallas_call(
        _body,
        grid=grid,
        in_specs=[
            pl.BlockSpec(
                (1, CHUNK, D_MODEL_K),
                lambda b, p, t: (b, jnp.where(p == 0, t, NT - 1), 0)),
            pl.BlockSpec((D_MODEL_K, ROUTER_HIDDEN_K), lambda b, p, t: (0, 0)),
            pl.BlockSpec((1, ROUTER_HIDDEN_K), lambda b, p, t: (0, 0)),
            pl.BlockSpec((ROUTER_HIDDEN_K, N_EXPERTS_K), lambda b, p, t: (0, 0)),
            pl.BlockSpec((1, N_EXPERTS_K), lambda b, p, t: (0, 0)),
            pl.BlockSpec((N_EXPERTS_K, D_MODEL_K), lambda b, p, t: (0, 0)),
            pl.BlockSpec((1, 1), lambda b, p, t: (0, 0)),
        ],
        out_specs=pl.BlockSpec(
            (1, CHUNK, D_MODEL_K),
            lambda b, p, t: (b, jnp.where(p == 1, t, 0), 0)),
        out_shape=jax.ShapeDtypeStruct((B_K, S_K, D_MODEL_K), jnp.float32),
        scratch_shapes=[
            pltpu.VMEM((S_K, D_MODEL_K), jnp.float32),
            pltpu.VMEM((1, D_MODEL_K), jnp.float32),
            pltpu.VMEM((1, D_MODEL_K), jnp.float32),
        ],
    )(x, W1, b1.reshape(1, -1), W2, b2.reshape(1, -1),
      expert_biases, bias_scale.reshape(1, 1))


def kernel(x, W1, b1, W2, b2, expert_biases, bias_scale):
    return _run(x, W1, b1, W2, b2, expert_biases, bias_scale)


# speedup vs baseline: 1.0589x; 1.0076x over previous
import jax
import jax.numpy as jnp
from jax.experimental import pallas as pl
from jax.experimental.pallas import tpu as pltpu

D_MODEL_K = 2048
N_EXPERTS_K = 8
ROUTER_HIDDEN_K = 64
B_K, S_K = 4, 4096
CHUNK = 512
NT = S_K // CHUNK


def _body(x_ref, W1_ref, b1_ref, W2_ref, b2_ref, eb_ref, scale_ref,
          out_ref, xbuf, acc, biasbuf, sem):
    i = pl.program_id(0)
    t = pl.program_id(1)

    read_on = jnp.logical_and(i < B_K, t >= 1)
    write_on = jnp.logical_and(i >= 1, t < NT)

    @pl.when(read_on)
    def _start_copy():
        pltpu.make_async_copy(
            x_ref.at[0], xbuf.at[pl.ds((t - 1) * CHUNK, CHUNK), :],
            sem).start()

    @pl.when(write_on)
    def _write():
        out_ref[0] = xbuf[pl.ds(t * CHUNK, CHUNK), :] + biasbuf[...]

    @pl.when(read_on)
    def _sum():
        partial = jnp.sum(x_ref[0], axis=0, keepdims=True)

        @pl.when(t == 1)
        def _():
            acc[...] = partial

        @pl.when(t > 1)
        def _():
            acc[...] = acc[...] + partial

        pltpu.make_async_copy(
            x_ref.at[0], xbuf.at[pl.ds((t - 1) * CHUNK, CHUNK), :],
            sem).wait()

    @pl.when(jnp.logical_and(i < B_K, t == NT))
    def _router():
        mean = acc[...] * (1.0 / S_K)
        h = jnp.dot(mean, W1_ref[...],
                    preferred_element_type=jnp.float32) + b1_ref[...]
        h = jnp.maximum(h, 0.0)
        logits = jnp.dot(h, W2_ref[...],
                         preferred_element_type=jnp.float32) + b2_ref[...]
        idx = jax.lax.broadcasted_iota(jnp.int32, (1, N_EXPERTS_K), 1)
        m1 = jnp.max(logits, axis=1, keepdims=True)
        i1 = jnp.min(jnp.where(logits == m1, idx, N_EXPERTS_K),
                     axis=1, keepdims=True)
        l2 = jnp.where(idx == i1, -1e30, logits)
        m2 = jnp.max(l2, axis=1, keepdims=True)
        i2 = jnp.min(jnp.where(l2 == m2, idx, N_EXPERTS_K),
                     axis=1, keepdims=True)
        e2 = jnp.exp(m2 - m1)
        denom = 1.0 + e2
        wvec = jnp.where(idx == i1, 1.0 / denom,
                         jnp.where(idx == i2, e2 / denom, 0.0))
        comb = jnp.dot(wvec, eb_ref[...],
                       preferred_element_type=jnp.float32)
        biasbuf[...] = comb * scale_ref[...]


def _x_index(i, t):
    b = jnp.minimum(i, B_K - 1)
    c = jnp.where(i < B_K, jnp.maximum(t - 1, 0), NT - 1)
    return (b, c, 0)


def _out_index(i, t):
    b = jnp.maximum(i - 1, 0)
    c = jnp.where(i >= 1, jnp.minimum(t, NT - 1), 0)
    return (b, c, 0)


@jax.jit
def _run(x, W1, b1, W2, b2, expert_biases, bias_scale):
    grid = (B_K + 1, NT + 1)
    return pl.pallas_call(
        _body,
        grid=grid,
        in_specs=[
            pl.BlockSpec((1, CHUNK, D_MODEL_K), _x_index),
            pl.BlockSpec((D_MODEL_K, ROUTER_HIDDEN_K), lambda i, t: (0, 0)),
            pl.BlockSpec((1, ROUTER_HIDDEN_K), lambda i, t: (0, 0)),
            pl.BlockSpec((ROUTER_HIDDEN_K, N_EXPERTS_K), lambda i, t: (0, 0)),
            pl.BlockSpec((1, N_EXPERTS_K), lambda i, t: (0, 0)),
            pl.BlockSpec((N_EXPERTS_K, D_MODEL_K), lambda i, t: (0, 0)),
            pl.BlockSpec((1, 1), lambda i, t: (0, 0)),
        ],
        out_specs=pl.BlockSpec((1, CHUNK, D_MODEL_K), _out_index),
        out_shape=jax.ShapeDtypeStruct((B_K, S_K, D_MODEL_K), jnp.float32),
        scratch_shapes=[
            pltpu.VMEM((S_K, D_MODEL_K), jnp.float32),
            pltpu.VMEM((1, D_MODEL_K), jnp.float32),
            pltpu.VMEM((1, D_MODEL_K), jnp.float32),
            pltpu.SemaphoreType.DMA,
        ],
    )(x, W1, b1.reshape(1, -1), W2, b2.reshape(1, -1),
      expert_biases, bias_scale.reshape(1, 1))


def kernel(x, W1, b1, W2, b2, expert_biases, bias_scale):
    return _run(x, W1, b1, W2, b2, expert_biases, bias_scale)


# speedup vs baseline: 1.0810x; 1.0209x over previous
import jax
import jax.numpy as jnp
from jax.experimental import pallas as pl
from jax.experimental.pallas import tpu as pltpu

D_MODEL_K = 2048
N_EXPERTS_K = 8
ROUTER_HIDDEN_K = 64
B_K, S_K = 4, 4096
CHUNK = 512
NT = S_K // CHUNK


def _body(x_ref, W1_ref, b1_ref, W2_ref, b2_ref, eb_ref, scale_ref,
          out_ref, xbuf, acc, biasbuf):
    i = pl.program_id(0)
    t = pl.program_id(1)

    read_on = jnp.logical_and(i < B_K, t >= 1)
    write_on = jnp.logical_and(i >= 1, t < NT)

    @pl.when(write_on)
    def _write():
        out_ref[0] = xbuf[pl.ds(t * CHUNK, CHUNK), :] + biasbuf[...]

    @pl.when(read_on)
    def _sum():
        chunk = x_ref[0]
        xbuf[pl.ds((t - 1) * CHUNK, CHUNK), :] = chunk
        partial = jnp.sum(chunk, axis=0, keepdims=True)

        @pl.when(t == 1)
        def _():
            acc[...] = partial

        @pl.when(t > 1)
        def _():
            acc[...] = acc[...] + partial

    @pl.when(jnp.logical_and(i < B_K, t == NT))
    def _router():
        mean = acc[...] * (1.0 / S_K)
        h = jnp.dot(mean, W1_ref[...],
                    preferred_element_type=jnp.float32) + b1_ref[...]
        h = jnp.maximum(h, 0.0)
        logits = jnp.dot(h, W2_ref[...],
                         preferred_element_type=jnp.float32) + b2_ref[...]
        idx = jax.lax.broadcasted_iota(jnp.int32, (1, N_EXPERTS_K), 1)
        m1 = jnp.max(logits, axis=1, keepdims=True)
        i1 = jnp.min(jnp.where(logits == m1, idx, N_EXPERTS_K),
                     axis=1, keepdims=True)
        l2 = jnp.where(idx == i1, -1e30, logits)
        m2 = jnp.max(l2, axis=1, keepdims=True)
        i2 = jnp.min(jnp.where(l2 == m2, idx, N_EXPERTS_K),
                     axis=1, keepdims=True)
        e2 = jnp.exp(m2 - m1)
        denom = 1.0 + e2
        wvec = jnp.where(idx == i1, 1.0 / denom,
                         jnp.where(idx == i2, e2 / denom, 0.0))
        comb = jnp.dot(wvec, eb_ref[...],
                       preferred_element_type=jnp.float32)
        biasbuf[...] = comb * scale_ref[...]


def _x_index(i, t):
    b = jnp.minimum(i, B_K - 1)
    c = jnp.where(i < B_K, jnp.maximum(t - 1, 0), NT - 1)
    return (b, c, 0)


def _out_index(i, t):
    b = jnp.maximum(i - 1, 0)
    c = jnp.where(i >= 1, jnp.minimum(t, NT - 1), 0)
    return (b, c, 0)


@jax.jit
def _run(x, W1, b1, W2, b2, expert_biases, bias_scale):
    grid = (B_K + 1, NT + 1)
    return pl.pallas_call(
        _body,
        grid=grid,
        in_specs=[
            pl.BlockSpec((1, CHUNK, D_MODEL_K), _x_index),
            pl.BlockSpec((D_MODEL_K, ROUTER_HIDDEN_K), lambda i, t: (0, 0)),
            pl.BlockSpec((1, ROUTER_HIDDEN_K), lambda i, t: (0, 0)),
            pl.BlockSpec((ROUTER_HIDDEN_K, N_EXPERTS_K), lambda i, t: (0, 0)),
            pl.BlockSpec((1, N_EXPERTS_K), lambda i, t: (0, 0)),
            pl.BlockSpec((N_EXPERTS_K, D_MODEL_K), lambda i, t: (0, 0)),
            pl.BlockSpec((1, 1), lambda i, t: (0, 0)),
        ],
        out_specs=pl.BlockSpec((1, CHUNK, D_MODEL_K), _out_index),
        out_shape=jax.ShapeDtypeStruct((B_K, S_K, D_MODEL_K), jnp.float32),
        scratch_shapes=[
            pltpu.VMEM((S_K, D_MODEL_K), jnp.float32),
            pltpu.VMEM((1, D_MODEL_K), jnp.float32),
            pltpu.VMEM((1, D_MODEL_K), jnp.float32),
        ],
    )(x, W1, b1.reshape(1, -1), W2, b2.reshape(1, -1),
      expert_biases, bias_scale.reshape(1, 1))


def kernel(x, W1, b1, W2, b2, expert_biases, bias_scale):
    return _run(x, W1, b1, W2, b2, expert_biases, bias_scale)
